# baseline (speedup 1.0000x reference)
import functools

import jax
import jax.numpy as jnp
from jax.experimental import pallas as pl
from jax.experimental.pallas import tpu as pltpu
from jax.experimental.pallas import tpu_sc as plsc

_KCODES = 8192
_DIM = 256
_TN = 256
_GW = 128


def _b2_body(cb_ref, b2_ref):
    cb = cb_ref[...]
    b2_ref[...] = jnp.sum(cb * cb, axis=1)[None, :]


def _vq_body(x_ref, cb_ref, b2_ref, idx_ref, oh_ref):
    x = x_ref[...]
    cb = cb_ref[...]
    a2 = jnp.sum(x * x, axis=1, keepdims=True)
    b2 = b2_ref[...]
    nab2 = jax.lax.dot_general(
        x * (-2.0), cb, (((1,), (1,)), ((), ())),
        preferred_element_type=jnp.float32)
    d = (a2 + b2) + nab2
    m = jnp.min(d, axis=1, keepdims=True)
    iota = jax.lax.broadcasted_iota(jnp.int32, d.shape, 1)
    masked = jnp.where(d == m, iota, jnp.int32(_KCODES))
    loc = jnp.min(masked, axis=1, keepdims=True)
    oh_ref[...] = (iota == loc).astype(jnp.float32)
    idx_ref[...] = loc


def _vq_body_alias(x_ref, cb_ref, b2_ref, oh_in_ref, idx_ref, oh_ref):
    del oh_in_ref
    _vq_body(x_ref, cb_ref, b2_ref, idx_ref, oh_ref)


def _sc_gather(codebook, idx_flat, n):
    num_workers = 32
    b_per_w = n // num_workers
    mesh = plsc.VectorSubcoreMesh(core_axis_name="c", subcore_axis_name="s")

    @functools.partial(
        pl.kernel, mesh=mesh,
        out_type=jax.ShapeDtypeStruct((n, _DIM), jnp.float32),
        scratch_types=[
            pltpu.VMEM((b_per_w,), jnp.int32),
            pltpu.VMEM((b_per_w, _DIM), jnp.float32),
            pltpu.SemaphoreType.DMA,
        ],
    )
    def gather_kernel(cb_hbm, idx_hbm, out_hbm, idx_v, rows_v, sem):
        wid = jax.lax.axis_index("s") * 2 + jax.lax.axis_index("c")
        base = wid * b_per_w
        pltpu.sync_copy(idx_hbm.at[pl.ds(base, b_per_w)], idx_v)
        pltpu.async_copy(cb_hbm.at[idx_v], rows_v, sem).wait()
        pltpu.sync_copy(rows_v, out_hbm.at[pl.ds(base, b_per_w)])

    return gather_kernel(codebook, idx_flat)


def kernel(x, codebook):
    b, t, d = x.shape
    n = b * t
    xf = x.reshape(n, d)
    b2 = pl.pallas_call(
        _b2_body,
        out_shape=jax.ShapeDtypeStruct((1, _KCODES), jnp.float32),
    )(codebook)
    nh = n // 2
    hb = nh // _TN

    def _half(base_blk, oh_in):
        body = _vq_body if oh_in is None else _vq_body_alias
        in_specs = [
            pl.BlockSpec((_TN, d), lambda i: (i + base_blk, 0)),
            pl.BlockSpec((_KCODES, d), lambda i: (0, 0)),
            pl.BlockSpec((1, _KCODES), lambda i: (0, 0)),
        ]
        operands = [xf, codebook, b2]
        kwargs = {}
        if oh_in is not None:
            in_specs.append(pl.BlockSpec(memory_space=pl.ANY))
            operands.append(oh_in)
            kwargs["input_output_aliases"] = {3: 1}
        return pl.pallas_call(
            body,
            grid=(hb,),
            in_specs=in_specs,
            out_specs=[
                pl.BlockSpec((_TN, 1), lambda i: (i, 0)),
                pl.BlockSpec((_TN, _KCODES), lambda i: (i + base_blk, 0)),
            ],
            out_shape=[
                jax.ShapeDtypeStruct((nh, 1), jnp.int32),
                jax.ShapeDtypeStruct((n, _KCODES), jnp.float32),
            ],
            **kwargs,
        )(*operands)

    idx0, oh_p = _half(0, None)
    q0 = _sc_gather(codebook, idx0.reshape(nh), nh)
    idx1, oh = _half(hb, oh_p)
    q1 = _sc_gather(codebook, idx1.reshape(nh), nh)
    q = jnp.concatenate([q0, q1], axis=0)
    idx = jnp.concatenate([idx0, idx1], axis=0)
    return (q.reshape(b, t, d), idx.reshape(b, t), oh.reshape(b, t, _KCODES))


# speedup vs baseline: 1.0990x; 1.0990x over previous
import functools

import jax
import jax.numpy as jnp
from jax.experimental import pallas as pl
from jax.experimental.pallas import tpu as pltpu
from jax.experimental.pallas import tpu_sc as plsc

_KCODES = 8192
_DIM = 256
_TN = 256
_GW = 128


def _b2_body(cb_ref, b2_ref):
    cb = cb_ref[...]
    b2_ref[...] = jnp.sum(cb * cb, axis=1, keepdims=True)


def _vq_body(x_ref, cb_ref, b2_ref, idx_ref, oh_ref):
    x = x_ref[...]
    cb = cb_ref[...]
    a2 = jnp.sum(x * x, axis=1, keepdims=True)
    b2 = b2_ref[...]
    nab2 = jax.lax.dot_general(
        x * (-2.0), cb, (((1,), (1,)), ((), ())),
        preferred_element_type=jnp.float32)
    d = (a2 + b2) + nab2
    m = jnp.min(d, axis=1, keepdims=True)
    iota = jax.lax.broadcasted_iota(jnp.int32, d.shape, 1)
    masked = jnp.where(d == m, iota, jnp.int32(_KCODES))
    loc = jnp.min(masked, axis=1, keepdims=True)
    oh_ref[...] = (iota == loc).astype(jnp.float32)
    idx_ref[...] = loc


def _sc_gather(codebook, idx_flat, n):
    num_workers = 32
    b_per_w = n // num_workers
    mesh = plsc.VectorSubcoreMesh(core_axis_name="c", subcore_axis_name="s")

    @functools.partial(
        pl.kernel, mesh=mesh,
        out_type=jax.ShapeDtypeStruct((n, _DIM), jnp.float32),
        scratch_types=[
            pltpu.VMEM((b_per_w,), jnp.int32),
            pltpu.VMEM((b_per_w, _DIM), jnp.float32),
            pltpu.SemaphoreType.DMA,
        ],
    )
    def gather_kernel(cb_hbm, idx_hbm, out_hbm, idx_v, rows_v, sem):
        wid = jax.lax.axis_index("s") * 2 + jax.lax.axis_index("c")
        base = wid * b_per_w
        pltpu.sync_copy(idx_hbm.at[pl.ds(base, b_per_w)], idx_v)
        pltpu.async_copy(cb_hbm.at[idx_v], rows_v, sem).wait()
        pltpu.sync_copy(rows_v, out_hbm.at[pl.ds(base, b_per_w)])

    return gather_kernel(codebook, idx_flat)


def kernel(x, codebook):
    b, t, d = x.shape
    n = b * t
    xf = x.reshape(n, d)
    b2 = pl.pallas_call(
        _b2_body,
        out_shape=jax.ShapeDtypeStruct((_KCODES, 1), jnp.float32),
    )(codebook).reshape(1, _KCODES)
    idx, oh = pl.pallas_call(
        _vq_body,
        grid=(n // _TN,),
        in_specs=[
            pl.BlockSpec((_TN, d), lambda i: (i, 0)),
            pl.BlockSpec((_KCODES, d), lambda i: (0, 0)),
            pl.BlockSpec((1, _KCODES), lambda i: (0, 0)),
        ],
        out_specs=[
            pl.BlockSpec((_TN, 1), lambda i: (i, 0)),
            pl.BlockSpec((_TN, _KCODES), lambda i: (i, 0)),
        ],
        out_shape=[
            jax.ShapeDtypeStruct((n, 1), jnp.int32),
            jax.ShapeDtypeStruct((n, _KCODES), jnp.float32),
        ],
    )(xf, codebook, b2)
    q = _sc_gather(codebook, idx.reshape(n), n)
    return (q.reshape(b, t, d), idx.reshape(b, t), oh.reshape(b, t, _KCODES))


# speedup vs baseline: 1.1179x; 1.0172x over previous
import functools

import jax
import jax.numpy as jnp
from jax.experimental import pallas as pl
from jax.experimental.pallas import tpu as pltpu
from jax.experimental.pallas import tpu_sc as plsc

_KCODES = 8192
_DIM = 256
_TN = 256
_GW = 128


def _b2_body(cb_ref, b2_ref):
    cb = cb_ref[...]
    b2_ref[...] = jnp.sum(cb * cb, axis=1)[None, :]


def _vq_body(x_ref, cb_ref, b2_ref, idx_ref, oh_ref):
    x = x_ref[...]
    cb = cb_ref[...]
    a2 = jnp.sum(x * x, axis=1, keepdims=True)
    b2 = b2_ref[...]
    nab2 = jax.lax.dot_general(
        x * (-2.0), cb, (((1,), (1,)), ((), ())),
        preferred_element_type=jnp.float32)
    d = (a2 + b2) + nab2
    m = jnp.min(d, axis=1, keepdims=True)
    iota = jax.lax.broadcasted_iota(jnp.int32, d.shape, 1)
    masked = jnp.where(d == m, iota, jnp.int32(_KCODES))
    loc = jnp.min(masked, axis=1, keepdims=True)
    oh_ref[...] = (iota == loc).astype(jnp.float32)
    idx_ref[...] = loc


def _sc_gather(codebook, idx_flat, n):
    num_workers = 32
    b_per_w = n // num_workers
    mesh = plsc.VectorSubcoreMesh(core_axis_name="c", subcore_axis_name="s")

    @functools.partial(
        pl.kernel, mesh=mesh,
        out_type=jax.ShapeDtypeStruct((n, _DIM), jnp.float32),
        scratch_types=[
            pltpu.VMEM((b_per_w,), jnp.int32),
            pltpu.VMEM((b_per_w, _DIM), jnp.float32),
            pltpu.SemaphoreType.DMA,
        ],
    )
    def gather_kernel(cb_hbm, idx_hbm, out_hbm, idx_v, rows_v, sem):
        wid = jax.lax.axis_index("s") * 2 + jax.lax.axis_index("c")
        base = wid * b_per_w
        pltpu.sync_copy(idx_hbm.at[pl.ds(base, b_per_w)], idx_v)
        pltpu.async_copy(cb_hbm.at[idx_v], rows_v, sem).wait()
        pltpu.sync_copy(rows_v, out_hbm.at[pl.ds(base, b_per_w)])

    return gather_kernel(codebook, idx_flat)


def kernel(x, codebook):
    b, t, d = x.shape
    n = b * t
    xf = x.reshape(n, d)
    b2 = pl.pallas_call(
        _b2_body,
        out_shape=jax.ShapeDtypeStruct((1, _KCODES), jnp.float32),
    )(codebook)
    idx, oh = pl.pallas_call(
        _vq_body,
        grid=(n // _TN,),
        in_specs=[
            pl.BlockSpec((_TN, d), lambda i: (i, 0)),
            pl.BlockSpec((_KCODES, d), lambda i: (0, 0)),
            pl.BlockSpec((1, _KCODES), lambda i: (0, 0)),
        ],
        out_specs=[
            pl.BlockSpec((_TN, 1), lambda i: (i, 0)),
            pl.BlockSpec((_TN, _KCODES), lambda i: (i, 0)),
        ],
        out_shape=[
            jax.ShapeDtypeStruct((n, 1), jnp.int32),
            jax.ShapeDtypeStruct((n, _KCODES), jnp.float32),
        ],
    )(xf, codebook, b2)
    q = _sc_gather(codebook, idx.reshape(n), n)
    return (q.reshape(b, t, d), idx.reshape(b, t), oh.reshape(b, t, _KCODES))


# speedup vs baseline: 1.1189x; 1.0009x over previous
import functools

import jax
import jax.numpy as jnp
from jax.experimental import pallas as pl
from jax.experimental.pallas import tpu as pltpu
from jax.experimental.pallas import tpu_sc as plsc

_KCODES = 8192
_DIM = 256
_TN = 512
_GW = 128


def _b2_body(cb_ref, b2_ref):
    cb = cb_ref[...]
    b2_ref[...] = jnp.sum(cb * cb, axis=1)[None, :]


def _vq_body(x_ref, cb_ref, b2_ref, idx_ref, oh_ref):
    x = x_ref[...]
    cb = cb_ref[...]
    a2 = jnp.sum(x * x, axis=1, keepdims=True)
    b2 = b2_ref[...]
    nab2 = jax.lax.dot_general(
        x * (-2.0), cb, (((1,), (1,)), ((), ())),
        preferred_element_type=jnp.float32)
    d = (a2 + b2) + nab2
    m = jnp.min(d, axis=1, keepdims=True)
    iota = jax.lax.broadcasted_iota(jnp.int32, d.shape, 1)
    masked = jnp.where(d == m, iota, jnp.int32(_KCODES))
    loc = jnp.min(masked, axis=1, keepdims=True)
    oh_ref[...] = (iota == loc).astype(jnp.float32)
    idx_ref[...] = loc


def _sc_gather(codebook, idx_flat, n):
    num_workers = 32
    b_per_w = n // num_workers
    mesh = plsc.VectorSubcoreMesh(core_axis_name="c", subcore_axis_name="s")

    @functools.partial(
        pl.kernel, mesh=mesh,
        out_type=jax.ShapeDtypeStruct((n, _DIM), jnp.float32),
        scratch_types=[
            pltpu.VMEM((b_per_w,), jnp.int32),
            pltpu.VMEM((b_per_w, _DIM), jnp.float32),
            pltpu.SemaphoreType.DMA,
        ],
    )
    def gather_kernel(cb_hbm, idx_hbm, out_hbm, idx_v, rows_v, sem):
        wid = jax.lax.axis_index("s") * 2 + jax.lax.axis_index("c")
        base = wid * b_per_w
        pltpu.sync_copy(idx_hbm.at[pl.ds(base, b_per_w)], idx_v)
        pltpu.async_copy(cb_hbm.at[idx_v], rows_v, sem).wait()
        pltpu.sync_copy(rows_v, out_hbm.at[pl.ds(base, b_per_w)])

    return gather_kernel(codebook, idx_flat)


def kernel(x, codebook):
    b, t, d = x.shape
    n = b * t
    xf = x.reshape(n, d)
    b2 = pl.pallas_call(
        _b2_body,
        out_shape=jax.ShapeDtypeStruct((1, _KCODES), jnp.float32),
    )(codebook)
    idx, oh = pl.pallas_call(
        _vq_body,
        grid=(n // _TN,),
        in_specs=[
            pl.BlockSpec((_TN, d), lambda i: (i, 0)),
            pl.BlockSpec((_KCODES, d), lambda i: (0, 0)),
            pl.BlockSpec((1, _KCODES), lambda i: (0, 0)),
        ],
        out_specs=[
            pl.BlockSpec((_TN, 1), lambda i: (i, 0)),
            pl.BlockSpec((_TN, _KCODES), lambda i: (i, 0)),
        ],
        out_shape=[
            jax.ShapeDtypeStruct((n, 1), jnp.int32),
            jax.ShapeDtypeStruct((n, _KCODES), jnp.float32),
        ],
    )(xf, codebook, b2)
    q = _sc_gather(codebook, idx.reshape(n), n)
    return (q.reshape(b, t, d), idx.reshape(b, t), oh.reshape(b, t, _KCODES))
